# baseline (speedup 1.0000x reference)
import jax
import jax.numpy as jnp
import numpy as np
from jax import lax
from jax.experimental import pallas as pl
from jax.experimental.pallas import tpu as pltpu
from jax.experimental.pallas import tpu_sc as plsc

_B, _C, _H, _W, _G = 8, 96, 224, 224, 4
_BH = _H // _G
_NC, _NS = 2, 16
_NW = _NC * _NS
_PAIRS_PER_W = (_B * _G * _G) // _NW
_CC = 8
_NCC = _C // _CC
_NSTEPS = _PAIRS_PER_W * _NCC
_NBUF = 4
_LAG = 2
_NSUPER = _NSTEPS // _NBUF


def _packed_inv() -> tuple[list[int], list[int]]:
    rng = np.random.RandomState(0)
    perms = np.stack([rng.permutation(_G * _G) for _ in range(_B)], axis=0)
    inv = np.argsort(perms, axis=1)
    lo = [int(sum(int(v) << (4 * j) for j, v in enumerate(row[:8]))) for row in inv]
    hi = [int(sum(int(v) << (4 * j) for j, v in enumerate(row[8:]))) for row in inv]
    return lo, hi


_PACKED_LO, _PACKED_HI = _packed_inv()


def _sc_body(x_hbm, out_hbm, bufs, rsem, wsem):
    wid = lax.axis_index("s") * _NC + lax.axis_index("c")
    b = wid >> 2

    lo = jnp.uint32(_PACKED_LO[0])
    hi = jnp.uint32(_PACKED_HI[0])
    for bb in range(1, _B):
        lo = jnp.where(b == bb, jnp.uint32(_PACKED_LO[bb]), lo)
        hi = jnp.where(b == bb, jnp.uint32(_PACKED_HI[bb]), hi)

    def coords(g):
        k = g // _NCC
        cc = g % _NCC
        t = (wid & 3) * 4 + k
        sl = 4 * jnp.where(t < 8, t, 0).astype(jnp.uint32)
        sh_ = 4 * jnp.where(t < 8, 0, t - 8).astype(jnp.uint32)
        src = jnp.where(t < 8, lo >> sl, hi >> sh_).astype(jnp.int32) & 15
        c0 = b * _C + cc * _CC
        return c0, src >> 2, src & 3, t >> 2, t & 3

    def read_start(g, j):
        c0, sh, sw, _, _ = coords(g)
        pltpu.make_async_copy(
            x_hbm.at[pl.ds(c0, _CC), pl.ds(sh * _BH, _BH), pl.ds(sw * _BH, _BH)],
            bufs[j],
            rsem[j],
        ).start()

    def read_wait(j):
        pltpu.make_async_copy(
            x_hbm.at[pl.ds(0, _CC), pl.ds(0, _BH), pl.ds(0, _BH)], bufs[j], rsem[j]
        ).wait()

    def write_start(g, j):
        c0, _, _, th, tw = coords(g)
        pltpu.make_async_copy(
            bufs[j],
            out_hbm.at[pl.ds(c0, _CC), pl.ds(th * _BH, _BH), pl.ds(tw * _BH, _BH)],
            wsem[j],
        ).start()

    def write_wait(j):
        pltpu.make_async_copy(
            bufs[j], out_hbm.at[pl.ds(0, _CC), pl.ds(0, _BH), pl.ds(0, _BH)], wsem[j]
        ).wait()

    for j in range(_NBUF):
        read_start(j, j)
        if j >= _LAG:
            jd = j - _LAG
            read_wait(jd)
            write_start(jd, jd)

    def superstep(s, carry):
        for j in range(_NBUF):
            g = s * _NBUF + j
            write_wait(j)
            read_start(g, j)
            jd = (j - _LAG) % _NBUF
            read_wait(jd)
            write_start(g - _LAG, jd)
        return carry

    lax.fori_loop(1, _NSUPER, superstep, 0)

    last = (_NSUPER - 1) * _NBUF
    for j in range(_LAG, _NBUF):
        read_wait(j)
        write_start(last + j, j)
    for j in range(_NBUF):
        write_wait(j)


_sc_call = pl.kernel(
    _sc_body,
    out_type=jax.ShapeDtypeStruct((_B * _C, _H, _W), jnp.float32),
    mesh=plsc.VectorSubcoreMesh(core_axis_name="c", subcore_axis_name="s"),
    scratch_types=[
        [pltpu.VMEM((_CC, _BH, _BH), jnp.float32) for _ in range(_NBUF)],
        [pltpu.SemaphoreType.DMA for _ in range(_NBUF)],
        [pltpu.SemaphoreType.DMA for _ in range(_NBUF)],
    ],
    compiler_params=pltpu.CompilerParams(use_tc_tiling_on_sc=False),
)


def kernel(x):
    x3 = x.reshape(_B * _C, _H, _W)
    out3 = _sc_call(x3)
    return out3.reshape(_B, _C, _H, _W)
